# baseline (speedup 1.0000x reference)
import functools

import jax
import jax.numpy as jnp
import numpy as np
from jax import lax
from jax.experimental import pallas as pl
from jax.experimental.pallas import tpu as pltpu
from jax.experimental.pallas import tpu_sc as plsc

EMB = 1000000
DIM = 32
PADW = 128
BATCH = 16384
NF = 26

NC = 2
NS = 16
NW = NC * NS
BPW = BATCH // NW
CHUNK = 32
NCHUNK = BPW // CHUNK
IDX_PER_CHUNK = CHUNK * NF
NGATHER = 8
GLEN = IDX_PER_CHUNK // NGATHER

_U = np.uint32
_M_HI = _U(17179)
_M_LO = _U(56963)


def _mod1e6(h):
    a = h >> _U(16)
    b = h & _U(0xFFFF)
    mid1 = a * _M_LO
    mid2 = b * _M_HI
    lo = b * _M_LO
    t = (lo >> _U(16)) + (mid1 & _U(0xFFFF)) + (mid2 & _U(0xFFFF))
    hi = a * _M_HI + (mid1 >> _U(16)) + (mid2 >> _U(16)) + (t >> _U(16))
    q = hi >> _U(18)
    return h - q * _U(1000000)


def _hash16(xv, cj):
    h = xv * _U(2654435761) + cj
    h = (h ^ (h >> _U(15))) * _U(2246822519)
    h = h ^ (h >> _U(13))
    return _mod1e6(h).astype(jnp.int32)


@functools.partial(
    pl.kernel,
    out_type=jax.ShapeDtypeStruct((BATCH * NF, DIM), jnp.float32),
    mesh=plsc.VectorSubcoreMesh(core_axis_name="c", subcore_axis_name="s"),
    compiler_params=pltpu.CompilerParams(
        needs_layout_passes=False, use_tc_tiling_on_sc=False
    ),
    scratch_types=[
        pltpu.VMEM((BPW,), jnp.int32),
        pltpu.VMEM((NF,), jnp.int32),
        pltpu.VMEM((NGATHER, GLEN), jnp.int32),
        pltpu.VMEM((IDX_PER_CHUNK, PADW), jnp.float32),
        pltpu.SemaphoreType.DMA,
    ],
)
def _emb_lookup(x_hbm, fnum_hbm, table_hbm, out_hbm, x_v, f_v, idx_v, rows_v, sem):
    wid = lax.axis_index("s") * NC + lax.axis_index("c")
    base = wid * BPW
    pltpu.sync_copy(x_hbm.at[pl.ds(base, BPW)], x_v)
    pltpu.sync_copy(fnum_hbm, f_v)
    lane = lax.iota(jnp.int32, 16)

    for c in range(NCHUNK):
        def field_body(j, carry, c=c):
            fj = plsc.load_gather(f_v, [jnp.full((16,), j, jnp.int32)])
            cj = fj.astype(_U) * _U(40503) + _U(2166136261)
            for k in range(CHUNK // 16):
                xv = x_v[pl.ds(c * CHUNK + k * 16, 16)].astype(_U)
                idx = _hash16(xv, cj)
                p = (k * 16 + lane) * NF + j
                plsc.store_scatter(idx_v, [p // GLEN, p % GLEN], idx)
            return carry
        lax.fori_loop(0, NF, field_body, 0)

        copies = [
            pltpu.async_copy(
                table_hbm.at[idx_v.at[g]],
                rows_v.at[pl.ds(g * GLEN, GLEN)],
                sem,
            )
            for g in range(NGATHER)
        ]
        for cp in copies:
            cp.wait()
        out_base = (base + c * CHUNK) * NF
        pltpu.sync_copy(
            rows_v.at[:, pl.ds(0, DIM)],
            out_hbm.at[pl.ds(out_base, IDX_PER_CHUNK)],
        )


def kernel(x, fnum, table):
    tpad = jnp.pad(table, ((0, 0), (0, PADW - DIM)))
    out = _emb_lookup(x, fnum, tpad)
    return out.reshape(BATCH, NF * DIM)


# speedup vs baseline: 1.1711x; 1.1711x over previous
import functools

import jax
import jax.numpy as jnp
import numpy as np
from jax import lax
from jax.experimental import pallas as pl
from jax.experimental.pallas import tpu as pltpu
from jax.experimental.pallas import tpu_sc as plsc

EMB = 1000000
DIM = 32
BATCH = 16384
NF = 26

NC = 2
NS = 16
NW = NC * NS
BPW = BATCH // NW
CHUNK = 64
NCHUNK = BPW // CHUNK
IDX_PER_CHUNK = CHUNK * NF
NGATHER = IDX_PER_CHUNK // 128

_U = np.uint32
_M_HI = _U(17179)
_M_LO = _U(56963)


def _mod1e6(h):
    a = h >> _U(16)
    b = h & _U(0xFFFF)
    mid1 = a * _M_LO
    mid2 = b * _M_HI
    lo = b * _M_LO
    t = (lo >> _U(16)) + (mid1 & _U(0xFFFF)) + (mid2 & _U(0xFFFF))
    hi = a * _M_HI + (mid1 >> _U(16)) + (mid2 >> _U(16)) + (t >> _U(16))
    q = hi >> _U(18)
    return h - q * _U(1000000)


def _hash16(xv, cj):
    h = xv * _U(2654435761) + cj
    h = (h ^ (h >> _U(15))) * _U(2246822519)
    h = h ^ (h >> _U(13))
    return _mod1e6(h).astype(jnp.int32)


@functools.partial(
    pl.kernel,
    out_type=jax.ShapeDtypeStruct((BATCH * NF, DIM), jnp.float32),
    mesh=plsc.VectorSubcoreMesh(core_axis_name="c", subcore_axis_name="s"),
    compiler_params=pltpu.CompilerParams(
        needs_layout_passes=False, use_tc_tiling_on_sc=False
    ),
    scratch_types=[
        pltpu.VMEM((BPW,), jnp.int32),
        pltpu.VMEM((NF,), jnp.int32),
        pltpu.VMEM((2, NGATHER, 128), jnp.int32),
        pltpu.VMEM((2, IDX_PER_CHUNK, DIM), jnp.float32),
        pltpu.SemaphoreType.DMA,
        pltpu.SemaphoreType.DMA,
    ],
)
def _emb_lookup(
    x_hbm, fnum_hbm, table_hbm, out_hbm, x_v, f_v, idx_v, rows_v, gsem, osem
):
    wid = lax.axis_index("s") * NC + lax.axis_index("c")
    base = wid * BPW
    pltpu.sync_copy(x_hbm.at[pl.ds(base, BPW)], x_v)
    pltpu.sync_copy(fnum_hbm, f_v)
    lane = lax.iota(jnp.int32, 16)

    def hash_chunk(c):
        buf = c % 2

        def field_body(j, carry):
            fj = plsc.load_gather(f_v, [jnp.full((16,), j, jnp.int32)])
            cj = fj.astype(_U) * _U(40503) + _U(2166136261)
            for k in range(CHUNK // 16):
                xv = x_v[pl.ds(c * CHUNK + k * 16, 16)].astype(_U)
                idx = _hash16(xv, cj)
                p = (k * 16 + lane) * NF + j
                plsc.store_scatter(idx_v.at[buf], [p >> 7, p & 127], idx)
            return carry

        lax.fori_loop(0, NF, field_body, 0)

    def fire_gathers(c):
        buf = c % 2
        return [
            pltpu.async_copy(
                table_hbm.at[idx_v.at[buf, g]],
                rows_v.at[buf, pl.ds(g * 128, 128)],
                gsem,
            )
            for g in range(NGATHER)
        ]

    def start_out_write(c):
        buf = c % 2
        out_base = (base + c * CHUNK) * NF
        return pltpu.async_copy(
            rows_v.at[buf], out_hbm.at[pl.ds(out_base, IDX_PER_CHUNK)], osem
        )

    gathers = [None] * NCHUNK
    writes = [None] * NCHUNK
    for c in range(NCHUNK):
        hash_chunk(c)
        if c >= 1:
            for cp in gathers[c - 1]:
                cp.wait()
            writes[c - 1] = start_out_write(c - 1)
        if c >= 2:
            writes[c - 2].wait()
        gathers[c] = fire_gathers(c)
    for cp in gathers[NCHUNK - 1]:
        cp.wait()
    writes[NCHUNK - 2].wait()
    writes[NCHUNK - 1] = start_out_write(NCHUNK - 1)
    writes[NCHUNK - 1].wait()


def kernel(x, fnum, table):
    out = _emb_lookup(x, fnum, table)
    return out.reshape(BATCH, NF * DIM)
